# Initial kernel scaffold; baseline (speedup 1.0000x reference)
import jax
import jax.numpy as jnp
from jax.experimental import pallas as pl


def kernel(pos, batch, W1, b1, W2, b2, W3, b3, Wf1, bf1, Wf2a, bf2a, Wf2b, bf2b, Wf2c, bf2c):
    raise NotImplementedError("write your pallas kernel here")


# speedup vs baseline: 18.3086x; 18.3086x over previous
import functools

import jax
import jax.numpy as jnp
from jax import lax
from jax.experimental import pallas as pl

_B, _P, _K, _OUT = 16, 1024, 20, 40
_NEG = -3e38


def _edge_kernel(x_ref, wa_ref, wb_ref, b_ref, out_ref):
    x = x_ref[0]
    n2 = jnp.sum(x * x, axis=1, keepdims=True)
    gram = lax.dot_general(x, x, (((1,), (1,)), ((), ())),
                           preferred_element_type=jnp.float32)
    ones_row = jnp.ones((1, x.shape[1]), jnp.float32)
    n2r = lax.dot_general(ones_row, x * x, (((1,), (1,)), ((), ())),
                          preferred_element_type=jnp.float32)
    dist = n2 + n2r - 2.0 * gram

    wb = wb_ref[...]
    y = lax.dot_general(x, wb, (((1,), (0,)), ((), ())),
                        preferred_element_type=jnp.float32)
    c = lax.dot_general(x, wa_ref[...] - wb, (((1,), (0,)), ((), ())),
                        preferred_element_type=jnp.float32) + b_ref[...]

    col = lax.broadcasted_iota(jnp.int32, (_P, _P), 1)
    acc = jnp.full(y.shape, _NEG, jnp.float32)
    for _ in range(_K):
        rowmin = jnp.min(dist, axis=1, keepdims=True)
        eq = dist == rowmin
        aidx = jnp.min(jnp.where(eq, col, _P), axis=1, keepdims=True)
        sel = col == aidx
        onehot = jnp.where(sel, 1.0, 0.0).astype(jnp.float32)
        picked = lax.dot_general(onehot, y, (((1,), (0,)), ((), ())),
                                 preferred_element_type=jnp.float32)
        acc = jnp.maximum(acc, picked)
        dist = jnp.where(sel, float("inf"), dist)
    out_ref[0] = c + acc


def _edge_layer(x, wa, wb, b):
    d, o = wa.shape
    return pl.pallas_call(
        _edge_kernel,
        grid=(_B,),
        in_specs=[
            pl.BlockSpec((1, _P, d), lambda i: (i, 0, 0)),
            pl.BlockSpec((d, o), lambda i: (0, 0)),
            pl.BlockSpec((d, o), lambda i: (0, 0)),
            pl.BlockSpec((1, o), lambda i: (0, 0)),
        ],
        out_specs=pl.BlockSpec((1, _P, o), lambda i: (i, 0, 0)),
        out_shape=jax.ShapeDtypeStruct((_B, _P, o), jnp.float32),
    )(x, wa, wb, b)


def _fc1_kernel(x1_ref, x2_ref, x3_ref, wf_a_ref, wf_b_ref, wf_c_ref,
                bf_ref, out_ref):
    h = lax.dot_general(x1_ref[0], wf_a_ref[...], (((1,), (0,)), ((), ())),
                        preferred_element_type=jnp.float32)
    h += lax.dot_general(x2_ref[0], wf_b_ref[...], (((1,), (0,)), ((), ())),
                         preferred_element_type=jnp.float32)
    h += lax.dot_general(x3_ref[0], wf_c_ref[...], (((1,), (0,)), ((), ())),
                         preferred_element_type=jnp.float32)
    h += bf_ref[...]
    out_ref[0] = jnp.max(h, axis=0, keepdims=True)


def _head_kernel(g_ref, wa_ref, ba_ref, wb_ref, bb_ref, wc_ref, bc_ref,
                 out_ref):
    o1 = lax.dot_general(g_ref[...], wa_ref[...], (((1,), (0,)), ((), ())),
                         preferred_element_type=jnp.float32) + ba_ref[...]
    o1 = jnp.maximum(o1, 0.0)
    o2 = lax.dot_general(o1, wb_ref[...], (((1,), (0,)), ((), ())),
                         preferred_element_type=jnp.float32) + bb_ref[...]
    o2 = jnp.maximum(o2, 0.0)
    o3 = lax.dot_general(o2, wc_ref[...], (((1,), (0,)), ((), ())),
                         preferred_element_type=jnp.float32) + bc_ref[...]
    m = jnp.max(o3, axis=1, keepdims=True)
    shifted = o3 - m
    lse = jnp.log(jnp.sum(jnp.exp(shifted), axis=1, keepdims=True))
    out_ref[...] = shifted - lse


def kernel(pos, batch, W1, b1, W2, b2, W3, b3, Wf1, bf1, Wf2a, bf2a,
           Wf2b, bf2b, Wf2c, bf2c):
    del batch
    x0 = pos.reshape(_B, _P, 3)
    x1 = _edge_layer(x0, W1[:3], W1[3:], b1.reshape(1, -1))
    x2 = _edge_layer(x1, W2[:64], W2[64:], b2.reshape(1, -1))
    x3 = _edge_layer(x2, W3[:128], W3[128:], b3.reshape(1, -1))

    g = pl.pallas_call(
        _fc1_kernel,
        grid=(_B,),
        in_specs=[
            pl.BlockSpec((1, _P, 64), lambda i: (i, 0, 0)),
            pl.BlockSpec((1, _P, 128), lambda i: (i, 0, 0)),
            pl.BlockSpec((1, _P, 256), lambda i: (i, 0, 0)),
            pl.BlockSpec((64, 1024), lambda i: (0, 0)),
            pl.BlockSpec((128, 1024), lambda i: (0, 0)),
            pl.BlockSpec((256, 1024), lambda i: (0, 0)),
            pl.BlockSpec((1, 1024), lambda i: (0, 0)),
        ],
        out_specs=pl.BlockSpec((1, 1, 1024), lambda i: (i, 0, 0)),
        out_shape=jax.ShapeDtypeStruct((_B, 1, 1024), jnp.float32),
    )(x1, x2, x3, Wf1[:64], Wf1[64:192], Wf1[192:], bf1.reshape(1, -1))
    g = g.reshape(_B, 1024)

    return pl.pallas_call(
        _head_kernel,
        out_shape=jax.ShapeDtypeStruct((_B, _OUT), jnp.float32),
    )(g, Wf2a, bf2a.reshape(1, -1), Wf2b, bf2b.reshape(1, -1),
      Wf2c, bf2c.reshape(1, -1))
